# Initial kernel scaffold; baseline (speedup 1.0000x reference)
import jax
import jax.numpy as jnp
from jax.experimental import pallas as pl


def kernel(x, edge_index, W1, b1, W2, b2, W3, b3, attn_W, attn_b, fc_W, fc_b):
    raise NotImplementedError("write your pallas kernel here")


# speedup vs baseline: 20.5807x; 20.5807x over previous
import functools

import jax
import jax.numpy as jnp
from jax import lax
from jax.experimental import pallas as pl
from jax.experimental.pallas import tpu as pltpu
from jax.experimental.pallas import tpu_sc as plsc

NC = 2
NS = 16
NW = NC * NS
LANES = 16
C = 80


def _t1_body(degsum_ref, x_ref, w1_ref, dis_ref, g1_ref):
    dis = lax.rsqrt(degsum_ref[...])
    dis_ref[...] = dis
    h = jnp.dot(x_ref[...], w1_ref[...], preferred_element_type=jnp.float32)
    g1_ref[...] = dis * h


def _t2_body(n, sp_ref, g_ref, dis_ref, b_ref, wn_ref, gn_ref):
    dis = dis_ref[...]
    s = sp_ref[0][:n] + sp_ref[1][:n] + g_ref[...]
    h = jnp.maximum(dis * s + b_ref[...], 0.0)
    gn_ref[...] = dis * jnp.dot(h, wn_ref[...],
                                preferred_element_type=jnp.float32)


def _t3_body(n, sp_ref, g_ref, dis_ref, b_ref, aw_ref, ab_ref, fw_ref, fb_ref,
             out_ref, attn_ref):
    dis = dis_ref[...]
    s = sp_ref[0][:n] + sp_ref[1][:n] + g_ref[...]
    h = jnp.maximum(dis * s + b_ref[...], 0.0)
    a = jnp.dot(h, aw_ref[...], preferred_element_type=jnp.float32) + ab_ref[...]
    m = jnp.max(a)
    e = jnp.exp(a - m)
    attn = e / jnp.sum(e)
    attn_ref[...] = attn
    o = jnp.dot(h * attn, fw_ref[...],
                preferred_element_type=jnp.float32) + fb_ref[...]
    out_ref[...] = 1.0 / (1.0 + jnp.exp(-o))


def _deg_body(chd, dt, dst_ref, out_ref, acc, idx, ones, zbuf):
    c = lax.axis_index("c")
    s = lax.axis_index("s")

    @pl.when(c == 0)
    def _():
        for k in range(C // LANES):
            ones[pl.ds(k * LANES, LANES)] = jnp.ones((LANES,), jnp.float32)

        def zrow(i, carry):
            zbuf[pl.ds(i * LANES, LANES)] = jnp.zeros((LANES,), jnp.float32)
            return carry
        lax.fori_loop(0, dt // LANES, zrow, 0)
        pltpu.sync_copy(zbuf, acc.at[pl.ds(s * dt, dt)])
        plsc.subcore_barrier()

        pltpu.sync_copy(dst_ref.at[s], idx)

        def chunk(j, carry):
            pltpu.sync_copy(ones, acc.at[idx.at[j]], add=True)
            return carry
        lax.fori_loop(0, chd, chunk, 0)
        plsc.subcore_barrier()

        pltpu.sync_copy(acc.at[pl.ds(s * dt, dt)], zbuf)
        pltpu.sync_copy(zbuf, out_ref.at[pl.ds(s * dt, dt)])


def _spmm_body(ch, rt, rc, h, src_ref, dst_ref, g_ref, out_ref,
               acc, sidx, didx, gbuf, cbuf, sem):
    c = lax.axis_index("c")
    s = lax.axis_index("s")
    w = c * NS + s

    def zrow(i, carry):
        for k in range(h // LANES):
            cbuf[i, pl.ds(k * LANES, LANES)] = jnp.zeros((LANES,), jnp.float32)
        return carry
    lax.fori_loop(0, rc, zrow, 0)
    for t in range(rt // rc):
        pltpu.sync_copy(cbuf, acc.at[pl.ds(s * rt + t * rc, rc)])
    plsc.subcore_barrier()

    pltpu.sync_copy(src_ref.at[w], sidx)
    pltpu.sync_copy(dst_ref.at[w], didx)

    def chunk(j, carry):
        pltpu.async_copy(g_ref.at[sidx.at[j]], gbuf, sem).wait()
        pltpu.sync_copy(gbuf, acc.at[didx.at[j]], add=True)
        return carry
    lax.fori_loop(0, ch, chunk, 0)
    plsc.subcore_barrier()

    for t in range(rt // rc):
        pltpu.sync_copy(acc.at[pl.ds(s * rt + t * rc, rc)], cbuf)
        pltpu.sync_copy(cbuf, out_ref.at[c, pl.ds(s * rt + t * rc, rc)])


def kernel(x, edge_index, W1, b1, W2, b2, W3, b3, attn_W, attn_b, fc_W, fc_b):
    N, d_in = x.shape
    E = edge_index.shape[1]
    H = W1.shape[1]

    ch = E // NW // C
    chd = E // NS // C
    npad = ((N + 128 * NS - 1) // (128 * NS)) * (128 * NS)
    rt = npad // NS
    rc = 128
    dt = npad // NS

    src = edge_index[0].reshape(NW, ch, C)
    dst = edge_index[1].reshape(NW, ch, C)
    dst16 = edge_index[1].reshape(NS, chd, C)

    mesh = plsc.VectorSubcoreMesh(core_axis_name="c", subcore_axis_name="s")

    deg_call = pl.kernel(
        functools.partial(_deg_body, chd, dt),
        out_type=jax.ShapeDtypeStruct((npad,), jnp.float32),
        mesh=mesh,
        scratch_types=[
            pltpu.VMEM_SHARED((npad,), jnp.float32),
            pltpu.VMEM((chd, C), jnp.int32),
            pltpu.VMEM((C,), jnp.float32),
            pltpu.VMEM((dt,), jnp.float32),
        ],
    )
    spmm_call = pl.kernel(
        functools.partial(_spmm_body, ch, rt, rc, H),
        out_type=jax.ShapeDtypeStruct((NC, npad, H), jnp.float32),
        mesh=mesh,
        scratch_types=[
            pltpu.VMEM_SHARED((npad, H), jnp.float32),
            pltpu.VMEM((ch, C), jnp.int32),
            pltpu.VMEM((ch, C), jnp.int32),
            pltpu.VMEM((C, H), jnp.float32),
            pltpu.VMEM((rc, H), jnp.float32),
            pltpu.SemaphoreType.DMA,
        ],
        compiler_params=pltpu.CompilerParams(use_tc_tiling_on_sc=False),
    )

    deg_p = deg_call(dst16)
    degsum = (deg_p[:N] + 1.0)[:, None]

    t1 = pl.pallas_call(
        _t1_body,
        out_shape=(jax.ShapeDtypeStruct((N, 1), jnp.float32),
                   jax.ShapeDtypeStruct((N, H), jnp.float32)),
    )
    dis, g = t1(degsum, x, W1)

    t2 = pl.pallas_call(
        functools.partial(_t2_body, N),
        out_shape=jax.ShapeDtypeStruct((N, H), jnp.float32),
    )
    for b, wn in ((b1, W2), (b2, W3)):
        sp = spmm_call(src, dst, g)
        g = t2(sp, g, dis, b.reshape(1, H), wn)

    sp = spmm_call(src, dst, g)
    t3 = pl.pallas_call(
        functools.partial(_t3_body, N),
        out_shape=(jax.ShapeDtypeStruct((N, 1), jnp.float32),
                   jax.ShapeDtypeStruct((N, 1), jnp.float32)),
    )
    out, attn = t3(sp, g, dis, b3.reshape(1, H), attn_W,
                   attn_b.reshape(1, 1), fc_W, fc_b.reshape(1, 1))
    return out, attn


# speedup vs baseline: 35.6594x; 1.7327x over previous
import functools

import jax
import jax.numpy as jnp
from jax import lax
from jax.experimental import pallas as pl
from jax.experimental.pallas import tpu as pltpu
from jax.experimental.pallas import tpu_sc as plsc

NC = 2
NS = 16
NW = NC * NS
LANES = 16
C = 80


def _t1_body(degsum_ref, x_ref, w1_ref, dis_ref, g1_ref):
    dis = lax.rsqrt(degsum_ref[...])
    dis_ref[...] = dis
    h = jnp.dot(x_ref[...], w1_ref[...], preferred_element_type=jnp.float32)
    g1_ref[...] = dis * h


def _t2_body(n, sp_ref, g_ref, dis_ref, b_ref, wn_ref, gn_ref):
    dis = dis_ref[...]
    s = sp_ref[0][:n] + sp_ref[1][:n] + g_ref[...]
    h = jnp.maximum(dis * s + b_ref[...], 0.0)
    gn_ref[...] = dis * jnp.dot(h, wn_ref[...],
                                preferred_element_type=jnp.float32)


def _t3_body(n, sp_ref, g_ref, dis_ref, b_ref, aw_ref, ab_ref, fw_ref, fb_ref,
             out_ref, attn_ref):
    dis = dis_ref[...]
    s = sp_ref[0][:n] + sp_ref[1][:n] + g_ref[...]
    h = jnp.maximum(dis * s + b_ref[...], 0.0)
    a = jnp.dot(h, aw_ref[...], preferred_element_type=jnp.float32) + ab_ref[...]
    m = jnp.max(a)
    e = jnp.exp(a - m)
    attn = e / jnp.sum(e)
    attn_ref[...] = attn
    o = jnp.dot(h * attn, fw_ref[...],
                preferred_element_type=jnp.float32) + fb_ref[...]
    out_ref[...] = 1.0 / (1.0 + jnp.exp(-o))


def _deg_body(chd, dt, dst_ref, out_ref, acc, idx, ones, zbuf):
    c = lax.axis_index("c")
    s = lax.axis_index("s")

    @pl.when(c == 0)
    def _():
        for k in range(C // LANES):
            ones[pl.ds(k * LANES, LANES)] = jnp.ones((LANES,), jnp.float32)

        def zrow(i, carry):
            zbuf[pl.ds(i * LANES, LANES)] = jnp.zeros((LANES,), jnp.float32)
            return carry
        lax.fori_loop(0, dt // LANES, zrow, 0)
        pltpu.sync_copy(zbuf, acc.at[pl.ds(s * dt, dt)])
        plsc.subcore_barrier()

        pltpu.sync_copy(dst_ref.at[s], idx)

        def chunk(j, carry):
            pltpu.sync_copy(ones, acc.at[idx.at[j]], add=True)
            return carry
        lax.fori_loop(0, chd, chunk, 0)
        plsc.subcore_barrier()

        pltpu.sync_copy(acc.at[pl.ds(s * dt, dt)], zbuf)
        pltpu.sync_copy(zbuf, out_ref.at[pl.ds(s * dt, dt)])


def _spmm_body(ch, rt, rc, h, nbuf, src_ref, dst_ref, g_ref, out_ref,
               acc, sidx, didx, gbuf, cbuf, *sems):
    gsem, ssem = sems[:nbuf], sems[nbuf:]
    c = lax.axis_index("c")
    s = lax.axis_index("s")
    w = c * NS + s

    def zrow(i, carry):
        for k in range(h // LANES):
            cbuf[i, pl.ds(k * LANES, LANES)] = jnp.zeros((LANES,), jnp.float32)
        return carry
    lax.fori_loop(0, rc, zrow, 0)
    for t in range(rt // rc):
        pltpu.sync_copy(cbuf, acc.at[pl.ds(s * rt + t * rc, rc)])
    plsc.subcore_barrier()

    pltpu.sync_copy(src_ref.at[w], sidx)
    pltpu.sync_copy(dst_ref.at[w], didx)

    for b in range(nbuf):
        pltpu.async_copy(g_ref.at[sidx.at[b]], gbuf.at[b], gsem[b])

    def body(i, carry):
        for b in range(nbuf):
            j = i * nbuf + b
            pltpu.make_async_copy(g_ref.at[sidx.at[j]], gbuf.at[b],
                                  gsem[b]).wait()
            pltpu.async_copy(gbuf.at[b], acc.at[didx.at[j]], ssem[b],
                             add=True)
        for b in range(nbuf):
            j = i * nbuf + b
            pltpu.make_async_copy(gbuf.at[b], acc.at[didx.at[j]],
                                  ssem[b]).wait()
            jn = j + nbuf

            @pl.when(jn < ch)
            def _():
                pltpu.async_copy(g_ref.at[sidx.at[jn]], gbuf.at[b], gsem[b])
        return carry
    lax.fori_loop(0, ch // nbuf, body, 0)
    plsc.subcore_barrier()

    for t in range(rt // rc):
        pltpu.sync_copy(acc.at[pl.ds(s * rt + t * rc, rc)], cbuf)
        pltpu.sync_copy(cbuf, out_ref.at[c, pl.ds(s * rt + t * rc, rc)])


def kernel(x, edge_index, W1, b1, W2, b2, W3, b3, attn_W, attn_b, fc_W, fc_b):
    N, d_in = x.shape
    E = edge_index.shape[1]
    H = W1.shape[1]

    cs = 100
    ch = E // NW // cs
    chd = E // NS // C
    nbuf = 4
    npad = ((N + 128 * NS - 1) // (128 * NS)) * (128 * NS)
    rt = npad // NS
    rc = 128
    dt = npad // NS

    src = edge_index[0].reshape(NW, ch, cs)
    dst = edge_index[1].reshape(NW, ch, cs)
    dst16 = edge_index[1].reshape(NS, chd, C)

    mesh = plsc.VectorSubcoreMesh(core_axis_name="c", subcore_axis_name="s")

    deg_call = pl.kernel(
        functools.partial(_deg_body, chd, dt),
        out_type=jax.ShapeDtypeStruct((npad,), jnp.float32),
        mesh=mesh,
        scratch_types=[
            pltpu.VMEM_SHARED((npad,), jnp.float32),
            pltpu.VMEM((chd, C), jnp.int32),
            pltpu.VMEM((C,), jnp.float32),
            pltpu.VMEM((dt,), jnp.float32),
        ],
    )
    spmm_call = pl.kernel(
        functools.partial(_spmm_body, ch, rt, rc, H, nbuf),
        out_type=jax.ShapeDtypeStruct((NC, npad, H), jnp.float32),
        mesh=mesh,
        scratch_types=[
            pltpu.VMEM_SHARED((npad, H), jnp.float32),
            pltpu.VMEM((ch, cs), jnp.int32),
            pltpu.VMEM((ch, cs), jnp.int32),
            pltpu.VMEM((nbuf, cs, H), jnp.float32),
            pltpu.VMEM((rc, H), jnp.float32),
        ] + [pltpu.SemaphoreType.DMA] * (2 * nbuf),
        compiler_params=pltpu.CompilerParams(use_tc_tiling_on_sc=False),
    )

    deg_p = deg_call(dst16)
    degsum = (deg_p[:N] + 1.0)[:, None]

    t1 = pl.pallas_call(
        _t1_body,
        out_shape=(jax.ShapeDtypeStruct((N, 1), jnp.float32),
                   jax.ShapeDtypeStruct((N, H), jnp.float32)),
    )
    dis, g = t1(degsum, x, W1)

    t2 = pl.pallas_call(
        functools.partial(_t2_body, N),
        out_shape=jax.ShapeDtypeStruct((N, H), jnp.float32),
    )
    for b, wn in ((b1, W2), (b2, W3)):
        sp = spmm_call(src, dst, g)
        g = t2(sp, g, dis, b.reshape(1, H), wn)

    sp = spmm_call(src, dst, g)
    t3 = pl.pallas_call(
        functools.partial(_t3_body, N),
        out_shape=(jax.ShapeDtypeStruct((N, 1), jnp.float32),
                   jax.ShapeDtypeStruct((N, 1), jnp.float32)),
    )
    out, attn = t3(sp, g, dis, b3.reshape(1, H), attn_W,
                   attn_b.reshape(1, 1), fc_W, fc_b.reshape(1, 1))
    return out, attn


# speedup vs baseline: 39.5792x; 1.1099x over previous
import functools

import jax
import jax.numpy as jnp
from jax import lax
from jax.experimental import pallas as pl
from jax.experimental.pallas import tpu as pltpu
from jax.experimental.pallas import tpu_sc as plsc

NC = 2
NS = 16
NW = NC * NS
LANES = 16
C = 80


def _t1_body(degsum_ref, x_ref, w1_ref, dis_ref, g1_ref):
    dis = lax.rsqrt(degsum_ref[...])
    dis_ref[...] = dis
    h = jnp.dot(x_ref[...], w1_ref[...], preferred_element_type=jnp.float32)
    g1_ref[...] = dis * h


def _t2_body(n, sp_ref, g_ref, dis_ref, b_ref, wn_ref, gn_ref):
    dis = dis_ref[...]
    s = sp_ref[0][:n] + sp_ref[1][:n] + g_ref[...]
    h = jnp.maximum(dis * s + b_ref[...], 0.0)
    gn_ref[...] = dis * jnp.dot(h, wn_ref[...],
                                preferred_element_type=jnp.float32)


def _t3_body(n, sp_ref, g_ref, dis_ref, b_ref, aw_ref, ab_ref, fw_ref, fb_ref,
             out_ref, attn_ref):
    dis = dis_ref[...]
    s = sp_ref[0][:n] + sp_ref[1][:n] + g_ref[...]
    h = jnp.maximum(dis * s + b_ref[...], 0.0)
    a = jnp.dot(h, aw_ref[...], preferred_element_type=jnp.float32) + ab_ref[...]
    m = jnp.max(a)
    e = jnp.exp(a - m)
    attn = e / jnp.sum(e)
    attn_ref[...] = attn
    o = jnp.dot(h * attn, fw_ref[...],
                preferred_element_type=jnp.float32) + fb_ref[...]
    out_ref[...] = 1.0 / (1.0 + jnp.exp(-o))


def _deg_body(chd, dt, dst_ref, out_ref, acc, idx, ones, zbuf, *sems):
    c = lax.axis_index("c")
    s = lax.axis_index("s")

    @pl.when(c == 0)
    def _():
        for k in range(C // LANES):
            ones[pl.ds(k * LANES, LANES)] = jnp.ones((LANES,), jnp.float32)

        def zrow(i, carry):
            zbuf[pl.ds(i * LANES, LANES)] = jnp.zeros((LANES,), jnp.float32)
            return carry
        lax.fori_loop(0, dt // LANES, zrow, 0)
        pltpu.sync_copy(zbuf, acc.at[pl.ds(s * dt, dt)])
        pltpu.sync_copy(dst_ref.at[s], idx)
        plsc.subcore_barrier()

        nbuf = 5
        for b in range(nbuf):
            pltpu.async_copy(ones, acc.at[idx.at[b]], sems[b], add=True)

        def body(i, carry):
            for b in range(nbuf):
                j = i * nbuf + b
                pltpu.make_async_copy(ones, acc.at[idx.at[j]],
                                      sems[b]).wait()
                jn = j + nbuf

                @pl.when(jn < chd)
                def _():
                    pltpu.async_copy(ones, acc.at[idx.at[jn]], sems[b],
                                     add=True)
            return carry
        lax.fori_loop(0, chd // nbuf, body, 0)
        plsc.subcore_barrier()

        pltpu.sync_copy(acc.at[pl.ds(s * dt, dt)], zbuf)
        pltpu.sync_copy(zbuf, out_ref.at[pl.ds(s * dt, dt)])


def _spmm_body(ch, rt, rc, h, nbuf, src_ref, dst_ref, g_ref, out_ref,
               acc, sidx, didx, gbuf, cbuf, *sems):
    gsem, ssem = sems[:nbuf], sems[nbuf:]
    c = lax.axis_index("c")
    s = lax.axis_index("s")
    w = c * NS + s

    pltpu.async_copy(src_ref.at[w], sidx, gsem[0])
    pltpu.async_copy(dst_ref.at[w], didx, gsem[1])

    def zrow(i, carry):
        for k in range(h // LANES):
            cbuf[0, i, pl.ds(k * LANES, LANES)] = jnp.zeros((LANES,),
                                                            jnp.float32)
        return carry
    lax.fori_loop(0, rc, zrow, 0)
    for t in range(rt // rc):
        pltpu.sync_copy(cbuf.at[0], acc.at[pl.ds(s * rt + t * rc, rc)])
    pltpu.make_async_copy(src_ref.at[w], sidx, gsem[0]).wait()
    pltpu.make_async_copy(dst_ref.at[w], didx, gsem[1]).wait()
    plsc.subcore_barrier()

    for b in range(nbuf):
        pltpu.async_copy(g_ref.at[sidx.at[b]], gbuf.at[b], gsem[b])

    def body(i, carry):
        for b in range(nbuf):
            j = i * nbuf + b
            pltpu.make_async_copy(g_ref.at[sidx.at[j]], gbuf.at[b],
                                  gsem[b]).wait()
            pltpu.async_copy(gbuf.at[b], acc.at[didx.at[j]], ssem[b],
                             add=True)
        for b in range(nbuf):
            j = i * nbuf + b
            pltpu.make_async_copy(gbuf.at[b], acc.at[didx.at[j]],
                                  ssem[b]).wait()
            jn = j + nbuf

            @pl.when(jn < ch)
            def _():
                pltpu.async_copy(g_ref.at[sidx.at[jn]], gbuf.at[b], gsem[b])
        return carry
    lax.fori_loop(0, ch // nbuf, body, 0)
    plsc.subcore_barrier()

    nt = rt // rc

    def pull(t, sem):
        return pltpu.make_async_copy(
            acc.at[pl.ds(s * rt + t * rc, rc)], cbuf.at[t % 2], sem)

    def push(t, sem):
        return pltpu.make_async_copy(
            cbuf.at[t % 2], out_ref.at[c, pl.ds(s * rt + t * rc, rc)], sem)

    pull(0, gsem[0]).start()
    for t in range(nt):
        b = t % 2
        pull(t, gsem[b]).wait()
        push(t, ssem[b]).start()
        if t + 1 < nt:
            nb = (t + 1) % 2
            if t >= 1:
                push(t - 1, ssem[nb]).wait()
            pull(t + 1, gsem[nb]).start()
    if nt >= 2:
        push(nt - 2, ssem[(nt - 2) % 2]).wait()
    push(nt - 1, ssem[(nt - 1) % 2]).wait()


def kernel(x, edge_index, W1, b1, W2, b2, W3, b3, attn_W, attn_b, fc_W, fc_b):
    N, d_in = x.shape
    E = edge_index.shape[1]
    H = W1.shape[1]

    cs = 125
    ch = E // NW // cs
    chd = E // NS // C
    nbuf = 5
    npad = ((N + 128 * NS - 1) // (128 * NS)) * (128 * NS)
    rt = npad // NS
    rc = 128
    dt = npad // NS

    src = edge_index[0].reshape(NW, ch, cs)
    dst = edge_index[1].reshape(NW, ch, cs)
    dst16 = edge_index[1].reshape(NS, chd, C)

    mesh = plsc.VectorSubcoreMesh(core_axis_name="c", subcore_axis_name="s")

    deg_call = pl.kernel(
        functools.partial(_deg_body, chd, dt),
        out_type=jax.ShapeDtypeStruct((npad,), jnp.float32),
        mesh=mesh,
        scratch_types=[
            pltpu.VMEM_SHARED((npad,), jnp.float32),
            pltpu.VMEM((chd, C), jnp.int32),
            pltpu.VMEM((C,), jnp.float32),
            pltpu.VMEM((dt,), jnp.float32),
        ] + [pltpu.SemaphoreType.DMA] * 5,
    )
    spmm_call = pl.kernel(
        functools.partial(_spmm_body, ch, rt, rc, H, nbuf),
        out_type=jax.ShapeDtypeStruct((NC, npad, H), jnp.float32),
        mesh=mesh,
        scratch_types=[
            pltpu.VMEM_SHARED((npad, H), jnp.float32),
            pltpu.VMEM((ch, cs), jnp.int32),
            pltpu.VMEM((ch, cs), jnp.int32),
            pltpu.VMEM((nbuf, cs, H), jnp.float32),
            pltpu.VMEM((2, rc, H), jnp.float32),
        ] + [pltpu.SemaphoreType.DMA] * (2 * nbuf),
        compiler_params=pltpu.CompilerParams(use_tc_tiling_on_sc=False),
    )

    deg_p = deg_call(dst16)
    degsum = (deg_p[:N] + 1.0)[:, None]

    t1 = pl.pallas_call(
        _t1_body,
        out_shape=(jax.ShapeDtypeStruct((N, 1), jnp.float32),
                   jax.ShapeDtypeStruct((N, H), jnp.float32)),
    )
    dis, g = t1(degsum, x, W1)

    t2 = pl.pallas_call(
        functools.partial(_t2_body, N),
        out_shape=jax.ShapeDtypeStruct((N, H), jnp.float32),
    )
    for b, wn in ((b1, W2), (b2, W3)):
        sp = spmm_call(src, dst, g)
        g = t2(sp, g, dis, b.reshape(1, H), wn)

    sp = spmm_call(src, dst, g)
    t3 = pl.pallas_call(
        functools.partial(_t3_body, N),
        out_shape=(jax.ShapeDtypeStruct((N, 1), jnp.float32),
                   jax.ShapeDtypeStruct((N, 1), jnp.float32)),
    )
    out, attn = t3(sp, g, dis, b3.reshape(1, H), attn_W,
                   attn_b.reshape(1, 1), fc_W, fc_b.reshape(1, 1))
    return out, attn


# speedup vs baseline: 40.7527x; 1.0297x over previous
import functools

import jax
import jax.numpy as jnp
from jax import lax
from jax.experimental import pallas as pl
from jax.experimental.pallas import tpu as pltpu
from jax.experimental.pallas import tpu_sc as plsc

NC = 2
NS = 16
NW = NC * NS
LANES = 16
C = 80


def _t1_body(degsum_ref, x_ref, w1_ref, dis_ref, g1_ref):
    dis = lax.rsqrt(degsum_ref[...])
    dis_ref[...] = dis
    h = jnp.dot(x_ref[...], w1_ref[...], preferred_element_type=jnp.float32)
    g1_ref[...] = dis * h


def _t2_body(n, sp_ref, g_ref, dis_ref, b_ref, wn_ref, gn_ref):
    dis = dis_ref[...]
    s = sp_ref[0][:n] + sp_ref[1][:n] + g_ref[...]
    h = jnp.maximum(dis * s + b_ref[...], 0.0)
    gn_ref[...] = dis * jnp.dot(h, wn_ref[...],
                                preferred_element_type=jnp.float32)


def _t3_body(n, sp_ref, g_ref, dis_ref, b_ref, aw_ref, ab_ref, fw_ref, fb_ref,
             out_ref, attn_ref):
    dis = dis_ref[...]
    s = sp_ref[0][:n] + sp_ref[1][:n] + g_ref[...]
    h = jnp.maximum(dis * s + b_ref[...], 0.0)
    a = jnp.dot(h, aw_ref[...], preferred_element_type=jnp.float32) + ab_ref[...]
    m = jnp.max(a)
    e = jnp.exp(a - m)
    attn = e / jnp.sum(e)
    attn_ref[...] = attn
    o = jnp.dot(h * attn, fw_ref[...],
                preferred_element_type=jnp.float32) + fb_ref[...]
    out_ref[...] = 1.0 / (1.0 + jnp.exp(-o))


def _deg_body(ept, chd, dt, ei_ref, out_ref, acc, idx, ones, zbuf, *sems):
    c = lax.axis_index("c")
    s = lax.axis_index("s")

    @pl.when(c == 0)
    def _():
        pltpu.async_copy(
            ei_ref.at[1, pl.ds(pl.multiple_of(s * ept, 8), ept)], idx,
            sems[0])
        for k in range(C // LANES):
            ones[pl.ds(k * LANES, LANES)] = jnp.ones((LANES,), jnp.float32)

        def zrow(i, carry):
            zbuf[pl.ds(i * LANES, LANES)] = jnp.zeros((LANES,), jnp.float32)
            return carry
        lax.fori_loop(0, dt // LANES, zrow, 0)
        pltpu.sync_copy(zbuf, acc.at[pl.ds(s * dt, dt)])
        pltpu.make_async_copy(
            ei_ref.at[1, pl.ds(pl.multiple_of(s * ept, 8), ept)], idx,
            sems[0]).wait()
        plsc.subcore_barrier()

        nbuf = 5

        def islc(j):
            return idx.at[pl.ds(pl.multiple_of(j * C, 8), C)]

        for b in range(nbuf):
            pltpu.async_copy(ones, acc.at[islc(b)], sems[b], add=True)

        def body(i, carry):
            for b in range(nbuf):
                j = i * nbuf + b
                pltpu.make_async_copy(ones, acc.at[islc(j)], sems[b]).wait()
                jn = j + nbuf

                @pl.when(jn < chd)
                def _():
                    pltpu.async_copy(ones, acc.at[islc(jn)], sems[b],
                                     add=True)
            return carry
        lax.fori_loop(0, chd // nbuf, body, 0)
        plsc.subcore_barrier()

        pltpu.sync_copy(acc.at[pl.ds(s * dt, dt)], zbuf)
        pltpu.sync_copy(zbuf, out_ref.at[pl.ds(s * dt, dt)])


def _spmm_body(epw, cs, ch, rt, rc, h, nbuf, ei_ref, g_ref, out_ref,
               acc, sidx, didx, gbuf, cbuf, *sems):
    gsem, ssem = sems[:nbuf], sems[nbuf:]
    c = lax.axis_index("c")
    s = lax.axis_index("s")
    w = c * NS + s

    pltpu.async_copy(
        ei_ref.at[0, pl.ds(pl.multiple_of(w * epw, 8), epw)], sidx, gsem[0])
    pltpu.async_copy(
        ei_ref.at[1, pl.ds(pl.multiple_of(w * epw, 8), epw)], didx, gsem[1])

    def sslc(j):
        return sidx.at[pl.ds(pl.multiple_of(j * cs, 8), cs)]

    def dslc(j):
        return didx.at[pl.ds(pl.multiple_of(j * cs, 8), cs)]

    def zrow(i, carry):
        for k in range(h // LANES):
            cbuf[0, i, pl.ds(k * LANES, LANES)] = jnp.zeros((LANES,),
                                                            jnp.float32)
        return carry
    lax.fori_loop(0, rc, zrow, 0)
    for t in range(rt // rc):
        pltpu.sync_copy(cbuf.at[0], acc.at[pl.ds(s * rt + t * rc, rc)])
    pltpu.make_async_copy(
        ei_ref.at[0, pl.ds(pl.multiple_of(w * epw, 8), epw)], sidx,
        gsem[0]).wait()
    pltpu.make_async_copy(
        ei_ref.at[1, pl.ds(pl.multiple_of(w * epw, 8), epw)], didx,
        gsem[1]).wait()
    plsc.subcore_barrier()

    for b in range(nbuf):
        pltpu.async_copy(g_ref.at[sslc(b)], gbuf.at[b], gsem[b])

    def body(i, carry):
        for b in range(nbuf):
            j = i * nbuf + b
            pltpu.make_async_copy(g_ref.at[sslc(j)], gbuf.at[b],
                                  gsem[b]).wait()
            pltpu.async_copy(gbuf.at[b], acc.at[dslc(j)], ssem[b],
                             add=True)
        for b in range(nbuf):
            j = i * nbuf + b
            pltpu.make_async_copy(gbuf.at[b], acc.at[dslc(j)],
                                  ssem[b]).wait()
            jn = j + nbuf

            @pl.when(jn < ch)
            def _():
                pltpu.async_copy(g_ref.at[sslc(jn)], gbuf.at[b], gsem[b])
        return carry
    lax.fori_loop(0, ch // nbuf, body, 0)
    plsc.subcore_barrier()

    nt = rt // rc

    def pull(t, sem):
        return pltpu.make_async_copy(
            acc.at[pl.ds(s * rt + t * rc, rc)], cbuf.at[t % 2], sem)

    def push(t, sem):
        return pltpu.make_async_copy(
            cbuf.at[t % 2], out_ref.at[c, pl.ds(s * rt + t * rc, rc)], sem)

    pull(0, gsem[0]).start()
    for t in range(nt):
        b = t % 2
        pull(t, gsem[b]).wait()
        push(t, ssem[b]).start()
        if t + 1 < nt:
            nb = (t + 1) % 2
            if t >= 1:
                push(t - 1, ssem[nb]).wait()
            pull(t + 1, gsem[nb]).start()
    if nt >= 2:
        push(nt - 2, ssem[(nt - 2) % 2]).wait()
    push(nt - 1, ssem[(nt - 1) % 2]).wait()


def kernel(x, edge_index, W1, b1, W2, b2, W3, b3, attn_W, attn_b, fc_W, fc_b):
    N, d_in = x.shape
    E = edge_index.shape[1]
    H = W1.shape[1]

    epw = E // NW
    ept = E // NS
    cs = 80
    ch = epw // cs
    chd = ept // C
    nbuf = 5
    npad = ((N + 128 * NS - 1) // (128 * NS)) * (128 * NS)
    rt = npad // NS
    rc = 128
    dt = npad // NS

    mesh = plsc.VectorSubcoreMesh(core_axis_name="c", subcore_axis_name="s")

    deg_call = pl.kernel(
        functools.partial(_deg_body, ept, chd, dt),
        out_type=jax.ShapeDtypeStruct((npad,), jnp.float32),
        mesh=mesh,
        scratch_types=[
            pltpu.VMEM_SHARED((npad,), jnp.float32),
            pltpu.VMEM((ept,), jnp.int32),
            pltpu.VMEM((C,), jnp.float32),
            pltpu.VMEM((dt,), jnp.float32),
        ] + [pltpu.SemaphoreType.DMA] * 5,
        compiler_params=pltpu.CompilerParams(use_tc_tiling_on_sc=False),
    )
    spmm_call = pl.kernel(
        functools.partial(_spmm_body, epw, cs, ch, rt, rc, H, nbuf),
        out_type=jax.ShapeDtypeStruct((NC, npad, H), jnp.float32),
        mesh=mesh,
        scratch_types=[
            pltpu.VMEM_SHARED((npad, H), jnp.float32),
            pltpu.VMEM((epw,), jnp.int32),
            pltpu.VMEM((epw,), jnp.int32),
            pltpu.VMEM((nbuf, cs, H), jnp.float32),
            pltpu.VMEM((2, rc, H), jnp.float32),
        ] + [pltpu.SemaphoreType.DMA] * (2 * nbuf),
        compiler_params=pltpu.CompilerParams(use_tc_tiling_on_sc=False),
    )

    deg_p = deg_call(edge_index)
    degsum = (deg_p[:N] + 1.0)[:, None]

    t1 = pl.pallas_call(
        _t1_body,
        out_shape=(jax.ShapeDtypeStruct((N, 1), jnp.float32),
                   jax.ShapeDtypeStruct((N, H), jnp.float32)),
    )
    dis, g = t1(degsum, x, W1)

    t2 = pl.pallas_call(
        functools.partial(_t2_body, N),
        out_shape=jax.ShapeDtypeStruct((N, H), jnp.float32),
    )
    for b, wn in ((b1, W2), (b2, W3)):
        sp = spmm_call(edge_index, g)
        g = t2(sp, g, dis, b.reshape(1, H), wn)

    sp = spmm_call(edge_index, g)
    t3 = pl.pallas_call(
        functools.partial(_t3_body, N),
        out_shape=(jax.ShapeDtypeStruct((N, 1), jnp.float32),
                   jax.ShapeDtypeStruct((N, 1), jnp.float32)),
    )
    out, attn = t3(sp, g, dis, b3.reshape(1, H), attn_W,
                   attn_b.reshape(1, 1), fc_W, fc_b.reshape(1, 1))
    return out, attn


# speedup vs baseline: 47.3564x; 1.1620x over previous
import functools

import jax
import jax.numpy as jnp
from jax import lax
from jax.experimental import pallas as pl
from jax.experimental.pallas import tpu as pltpu
from jax.experimental.pallas import tpu_sc as plsc

NC = 2
NS = 16
NW = NC * NS
LANES = 16
C = 80


def _expander(h):
    lane = lax.broadcasted_iota(jnp.int32, (2, 2 * h), 1) // h
    sub = lax.broadcasted_iota(jnp.int32, (2, 2 * h), 0)
    return (lane == sub).astype(jnp.float32)


def _blockdiag(w):
    z = jnp.zeros_like(w)
    return jnp.concatenate(
        [jnp.concatenate([w, z], axis=1), jnp.concatenate([z, w], axis=1)],
        axis=0)


def _t1_body(n, h, degm_ref, x_ref, w1_ref, dis_ref, g1_ref):
    m = n // 2
    dis2 = lax.rsqrt(degm_ref[...] + 1.0)
    dis = jnp.dot(dis2, _expander(h), preferred_element_type=jnp.float32)
    dis_ref[...] = dis
    top = jnp.dot(x_ref[:m], w1_ref[...], preferred_element_type=jnp.float32)
    bot = jnp.dot(x_ref[m:], w1_ref[...], preferred_element_type=jnp.float32)
    g1_ref[...] = dis * jnp.concatenate([top, bot], axis=1)


def _sg(n, npad, sp_ref, g_ref, dis_ref, b_ref):
    m = n // 2
    rpad = npad // 2
    s = sp_ref[0:m] + sp_ref[rpad:rpad + m] + g_ref[...]
    b2 = jnp.concatenate([b_ref[...], b_ref[...]], axis=1)
    return jnp.maximum(dis_ref[...] * s + b2, 0.0)


def _t2_body(n, npad, sp_ref, g_ref, dis_ref, b_ref, wn_ref, gn_ref):
    hh = _sg(n, npad, sp_ref, g_ref, dis_ref, b_ref)
    wd = _blockdiag(wn_ref[...])
    gn_ref[...] = dis_ref[...] * jnp.dot(hh, wd,
                                         preferred_element_type=jnp.float32)


def _t3_body(n, npad, h, sp_ref, g_ref, dis_ref, b_ref, aw_ref, ab_ref,
             fw_ref, fb_ref, out_ref, attn_ref):
    hh = _sg(n, npad, sp_ref, g_ref, dis_ref, b_ref)
    awd = _blockdiag(aw_ref[...])
    a2 = jnp.dot(hh, awd, preferred_element_type=jnp.float32) + ab_ref[...]
    mx = jnp.max(a2)
    e = jnp.exp(a2 - mx)
    attn2 = e / jnp.sum(e)
    attn_ref[...] = attn2
    aex = jnp.dot(attn2, _expander(h), preferred_element_type=jnp.float32)
    fwd = _blockdiag(fw_ref[...])
    o = jnp.dot(hh * aex, fwd, preferred_element_type=jnp.float32) + fb_ref[...]
    out_ref[...] = 1.0 / (1.0 + jnp.exp(-o))


def _deg_body(ept, chd, dt, ei_ref, out_ref, acc, idx, ones, zbuf, *sems):
    c = lax.axis_index("c")
    s = lax.axis_index("s")

    @pl.when(c == 0)
    def _():
        pltpu.async_copy(
            ei_ref.at[1, pl.ds(pl.multiple_of(s * ept, 8), ept)], idx,
            sems[0])
        for k in range(C // LANES):
            ones[pl.ds(k * LANES, LANES)] = jnp.ones((LANES,), jnp.float32)

        def zrow(i, carry):
            zbuf[pl.ds(i * LANES, LANES)] = jnp.zeros((LANES,), jnp.float32)
            return carry
        lax.fori_loop(0, dt // LANES, zrow, 0)
        pltpu.sync_copy(zbuf, acc.at[pl.ds(s * dt, dt)])
        pltpu.make_async_copy(
            ei_ref.at[1, pl.ds(pl.multiple_of(s * ept, 8), ept)], idx,
            sems[0]).wait()
        plsc.subcore_barrier()

        nbuf = 5

        def islc(j):
            return idx.at[pl.ds(pl.multiple_of(j * C, 8), C)]

        for b in range(nbuf):
            pltpu.async_copy(ones, acc.at[islc(b)], sems[b], add=True)

        def body(i, carry):
            for b in range(nbuf):
                j = i * nbuf + b
                pltpu.make_async_copy(ones, acc.at[islc(j)], sems[b]).wait()
                jn = j + nbuf

                @pl.when(jn < chd)
                def _():
                    pltpu.async_copy(ones, acc.at[islc(jn)], sems[b],
                                     add=True)
            return carry
        lax.fori_loop(0, chd // nbuf, body, 0)
        plsc.subcore_barrier()

        pltpu.sync_copy(acc.at[pl.ds(s * dt, dt)], zbuf)
        pltpu.sync_copy(zbuf, out_ref.at[pl.ds(s * dt, dt)])


def _spmm_body(epw, cs, ch, rt, rc, h, nbuf, ei_ref, g_ref, out_ref,
               acc, sidx, didx, gbuf, cbuf, *sems):
    gsem, ssem = sems[:nbuf], sems[nbuf:]
    c = lax.axis_index("c")
    s = lax.axis_index("s")
    w = c * NS + s

    pltpu.async_copy(
        ei_ref.at[0, pl.ds(pl.multiple_of(w * epw, 8), epw)], sidx, gsem[0])
    pltpu.async_copy(
        ei_ref.at[1, pl.ds(pl.multiple_of(w * epw, 8), epw)], didx, gsem[1])

    def sslc(j):
        return sidx.at[pl.ds(pl.multiple_of(j * cs, 8), cs)]

    def dslc(j):
        return didx.at[pl.ds(pl.multiple_of(j * cs, 8), cs)]

    def zrow(i, carry):
        for k in range(h // LANES):
            cbuf[0, i, pl.ds(k * LANES, LANES)] = jnp.zeros((LANES,),
                                                            jnp.float32)
        return carry
    lax.fori_loop(0, rc, zrow, 0)
    for t in range(rt // rc):
        pltpu.sync_copy(cbuf.at[0], acc.at[pl.ds(s * rt + t * rc, rc)])
    pltpu.make_async_copy(
        ei_ref.at[0, pl.ds(pl.multiple_of(w * epw, 8), epw)], sidx,
        gsem[0]).wait()
    pltpu.make_async_copy(
        ei_ref.at[1, pl.ds(pl.multiple_of(w * epw, 8), epw)], didx,
        gsem[1]).wait()
    plsc.subcore_barrier()

    for b in range(nbuf):
        pltpu.async_copy(g_ref.at[sslc(b)], gbuf.at[b], gsem[b])

    def body(i, carry):
        for b in range(nbuf):
            j = i * nbuf + b
            pltpu.make_async_copy(g_ref.at[sslc(j)], gbuf.at[b],
                                  gsem[b]).wait()
            pltpu.async_copy(gbuf.at[b], acc.at[dslc(j)], ssem[b],
                             add=True)
        for b in range(nbuf):
            j = i * nbuf + b
            pltpu.make_async_copy(gbuf.at[b], acc.at[dslc(j)],
                                  ssem[b]).wait()
            jn = j + nbuf

            @pl.when(jn < ch)
            def _():
                pltpu.async_copy(g_ref.at[sslc(jn)], gbuf.at[b], gsem[b])
        return carry
    lax.fori_loop(0, ch // nbuf, body, 0)
    plsc.subcore_barrier()

    nt = rt // rc

    def pull(t, sem):
        return pltpu.make_async_copy(
            acc.at[pl.ds(s * rt + t * rc, rc)], cbuf.at[t % 2], sem)

    def push(t, sem):
        return pltpu.make_async_copy(
            cbuf.at[t % 2], out_ref.at[c, pl.ds(s * rt + t * rc, rc)], sem)

    pull(0, gsem[0]).start()
    for t in range(nt):
        b = t % 2
        pull(t, gsem[b]).wait()
        push(t, ssem[b]).start()
        if t + 1 < nt:
            nb = (t + 1) % 2
            if t >= 1:
                push(t - 1, ssem[nb]).wait()
            pull(t + 1, gsem[nb]).start()
    if nt >= 2:
        push(nt - 2, ssem[(nt - 2) % 2]).wait()
    push(nt - 1, ssem[(nt - 1) % 2]).wait()


def kernel(x, edge_index, W1, b1, W2, b2, W3, b3, attn_W, attn_b, fc_W, fc_b):
    N, d_in = x.shape
    E = edge_index.shape[1]
    H = W1.shape[1]

    epw = E // NW
    ept = E // NS
    cs = 80
    ch = epw // cs
    chd = ept // C
    nbuf = 5
    npad = ((N + 128 * NS - 1) // (128 * NS)) * (128 * NS)
    rt = npad // NS
    rc = 128
    dt = npad // NS

    mesh = plsc.VectorSubcoreMesh(core_axis_name="c", subcore_axis_name="s")

    deg_call = pl.kernel(
        functools.partial(_deg_body, ept, chd, dt),
        out_type=jax.ShapeDtypeStruct((npad,), jnp.float32),
        mesh=mesh,
        scratch_types=[
            pltpu.VMEM_SHARED((npad,), jnp.float32),
            pltpu.VMEM((ept,), jnp.int32),
            pltpu.VMEM((C,), jnp.float32),
            pltpu.VMEM((dt,), jnp.float32),
        ] + [pltpu.SemaphoreType.DMA] * 5,
        compiler_params=pltpu.CompilerParams(use_tc_tiling_on_sc=False),
    )
    spmm_call = pl.kernel(
        functools.partial(_spmm_body, epw, cs, ch, rt, rc, H, nbuf),
        out_type=jax.ShapeDtypeStruct((NC, npad, H), jnp.float32),
        mesh=mesh,
        scratch_types=[
            pltpu.VMEM_SHARED((npad, H), jnp.float32),
            pltpu.VMEM((epw,), jnp.int32),
            pltpu.VMEM((epw,), jnp.int32),
            pltpu.VMEM((nbuf, cs, H), jnp.float32),
            pltpu.VMEM((2, rc, H), jnp.float32),
        ] + [pltpu.SemaphoreType.DMA] * (2 * nbuf),
        compiler_params=pltpu.CompilerParams(use_tc_tiling_on_sc=False),
    )

    m = N // 2
    ei2 = (edge_index % m) * 2 + edge_index // m

    deg_p = deg_call(ei2)
    degm = deg_p[:N].reshape(m, 2)

    t1 = pl.pallas_call(
        functools.partial(_t1_body, N, H),
        out_shape=(jax.ShapeDtypeStruct((m, 2 * H), jnp.float32),
                   jax.ShapeDtypeStruct((m, 2 * H), jnp.float32)),
    )
    dis, g = t1(degm, x, W1)

    t2 = pl.pallas_call(
        functools.partial(_t2_body, N, npad),
        out_shape=jax.ShapeDtypeStruct((m, 2 * H), jnp.float32),
    )
    for b, wn in ((b1, W2), (b2, W3)):
        sp = spmm_call(ei2, g.reshape(N, H))
        g = t2(sp.reshape(NC * npad // 2, 2 * H), g, dis,
               b.reshape(1, H), wn)

    sp = spmm_call(ei2, g.reshape(N, H))
    t3 = pl.pallas_call(
        functools.partial(_t3_body, N, npad, H),
        out_shape=(jax.ShapeDtypeStruct((m, 2), jnp.float32),
                   jax.ShapeDtypeStruct((m, 2), jnp.float32)),
    )
    o2, a2 = t3(sp.reshape(NC * npad // 2, 2 * H), g, dis,
                b3.reshape(1, H), attn_W, attn_b.reshape(1, 1),
                fc_W, fc_b.reshape(1, 1))
    out = jnp.concatenate([o2[:, 0:1], o2[:, 1:2]], axis=0)
    attn = jnp.concatenate([a2[:, 0:1], a2[:, 1:2]], axis=0)
    return out, attn


# speedup vs baseline: 51.0151x; 1.0773x over previous
import functools

import jax
import jax.numpy as jnp
from jax import lax
from jax.experimental import pallas as pl
from jax.experimental.pallas import tpu as pltpu
from jax.experimental.pallas import tpu_sc as plsc

NC = 2
NS = 16
NW = NC * NS
LANES = 16
C = 80


def _expander(h):
    lane = lax.broadcasted_iota(jnp.int32, (2, 2 * h), 1) // h
    sub = lax.broadcasted_iota(jnp.int32, (2, 2 * h), 0)
    return (lane == sub).astype(jnp.float32)


def _blockdiag(w):
    z = jnp.zeros_like(w)
    return jnp.concatenate(
        [jnp.concatenate([w, z], axis=1), jnp.concatenate([z, w], axis=1)],
        axis=0)


def _t1_body(n, h, degm_ref, x_ref, w1_ref, dis_ref, g1_ref):
    m = n // 2
    dis2 = lax.rsqrt(degm_ref[...] + 1.0)
    dis = jnp.dot(dis2, _expander(h), preferred_element_type=jnp.float32)
    dis_ref[...] = dis
    top = jnp.dot(x_ref[:m], w1_ref[...], preferred_element_type=jnp.float32)
    bot = jnp.dot(x_ref[m:], w1_ref[...], preferred_element_type=jnp.float32)
    g1_ref[...] = dis * jnp.concatenate([top, bot], axis=1)


def _sg(n, npad, sp_ref, g_ref, dis_ref, b_ref):
    m = n // 2
    rpad = npad // 2
    s = sp_ref[0:m] + sp_ref[rpad:rpad + m] + g_ref[...]
    b2 = jnp.concatenate([b_ref[...], b_ref[...]], axis=1)
    return jnp.maximum(dis_ref[...] * s + b2, 0.0)


def _t2_body(n, npad, sp_ref, g_ref, dis_ref, b_ref, wn_ref, gn_ref):
    hh = _sg(n, npad, sp_ref, g_ref, dis_ref, b_ref)
    wd = _blockdiag(wn_ref[...])
    gn_ref[...] = dis_ref[...] * jnp.dot(hh, wd,
                                         preferred_element_type=jnp.float32)


def _t3_body(n, npad, h, sp_ref, g_ref, dis_ref, b_ref, aw_ref, ab_ref,
             fw_ref, fb_ref, out_ref, attn_ref):
    hh = _sg(n, npad, sp_ref, g_ref, dis_ref, b_ref)
    awd = _blockdiag(aw_ref[...])
    a2 = jnp.dot(hh, awd, preferred_element_type=jnp.float32) + ab_ref[...]
    mx = jnp.max(a2)
    e = jnp.exp(a2 - mx)
    attn2 = e / jnp.sum(e)
    attn_ref[...] = attn2
    aex = jnp.dot(attn2, _expander(h), preferred_element_type=jnp.float32)
    fwd = _blockdiag(fw_ref[...])
    o = jnp.dot(hh * aex, fwd, preferred_element_type=jnp.float32) + fb_ref[...]
    out_ref[...] = 1.0 / (1.0 + jnp.exp(-o))


def _deg_body(ne, ept, chd, dt, ei_ref, out_ref, acc, idx, ones, zbuf, *sems):
    c = lax.axis_index("c")
    s = lax.axis_index("s")

    @pl.when(c == 0)
    def _():
        pltpu.async_copy(
            ei_ref.at[pl.ds(pl.multiple_of(ne + s * ept, 8), ept)], idx,
            sems[0])
        for k in range(C // LANES):
            ones[pl.ds(k * LANES, LANES)] = jnp.ones((LANES,), jnp.float32)

        def zrow(i, carry):
            zbuf[pl.ds(i * LANES, LANES)] = jnp.zeros((LANES,), jnp.float32)
            return carry
        lax.fori_loop(0, dt // LANES, zrow, 0)
        pltpu.sync_copy(zbuf, acc.at[pl.ds(s * dt, dt)])
        pltpu.make_async_copy(
            ei_ref.at[pl.ds(pl.multiple_of(ne + s * ept, 8), ept)], idx,
            sems[0]).wait()
        plsc.subcore_barrier()

        nbuf = 5

        def islc(j):
            return idx.at[pl.ds(pl.multiple_of(j * C, 8), C)]

        for b in range(nbuf):
            pltpu.async_copy(ones, acc.at[islc(b)], sems[b], add=True)

        def body(i, carry):
            for b in range(nbuf):
                j = i * nbuf + b
                pltpu.make_async_copy(ones, acc.at[islc(j)], sems[b]).wait()
                jn = j + nbuf

                @pl.when(jn < chd)
                def _():
                    pltpu.async_copy(ones, acc.at[islc(jn)], sems[b],
                                     add=True)
            return carry
        lax.fori_loop(0, chd // nbuf, body, 0)
        plsc.subcore_barrier()

        pltpu.sync_copy(acc.at[pl.ds(s * dt, dt)], zbuf)
        pltpu.sync_copy(zbuf, out_ref.at[pl.ds(s * dt, dt)])


def _spmm_body(ne, epw, cs, ch, rt, rc, h, nbuf, ei_ref, g_ref, out_ref,
               acc, sidx, didx, gbuf, cbuf, *sems):
    gsem, ssem = sems[:nbuf], sems[nbuf:]
    c = lax.axis_index("c")
    s = lax.axis_index("s")
    w = c * NS + s

    pltpu.async_copy(
        ei_ref.at[pl.ds(pl.multiple_of(w * epw, 8), epw)], sidx, gsem[0])
    pltpu.async_copy(
        ei_ref.at[pl.ds(pl.multiple_of(ne + w * epw, 8), epw)], didx, gsem[1])

    def sslc(j):
        return sidx.at[pl.ds(pl.multiple_of(j * cs, 8), cs)]

    def dslc(j):
        return didx.at[pl.ds(pl.multiple_of(j * cs, 8), cs)]

    def zrow(i, carry):
        for k in range(h // LANES):
            cbuf[0, i, pl.ds(k * LANES, LANES)] = jnp.zeros((LANES,),
                                                            jnp.float32)
        return carry
    lax.fori_loop(0, rc, zrow, 0)
    for t in range(rt // rc):
        pltpu.sync_copy(cbuf.at[0], acc.at[pl.ds(s * rt + t * rc, rc)])
    pltpu.make_async_copy(
        ei_ref.at[pl.ds(pl.multiple_of(w * epw, 8), epw)], sidx,
        gsem[0]).wait()
    pltpu.make_async_copy(
        ei_ref.at[pl.ds(pl.multiple_of(ne + w * epw, 8), epw)], didx,
        gsem[1]).wait()
    plsc.subcore_barrier()

    for b in range(nbuf):
        pltpu.async_copy(g_ref.at[sslc(b)], gbuf.at[b], gsem[b])

    def body(i, carry):
        for b in range(nbuf):
            j = i * nbuf + b
            pltpu.make_async_copy(g_ref.at[sslc(j)], gbuf.at[b],
                                  gsem[b]).wait()
            pltpu.async_copy(gbuf.at[b], acc.at[dslc(j)], ssem[b],
                             add=True)
        for b in range(nbuf):
            j = i * nbuf + b
            pltpu.make_async_copy(gbuf.at[b], acc.at[dslc(j)],
                                  ssem[b]).wait()
            jn = j + nbuf

            @pl.when(jn < ch)
            def _():
                pltpu.async_copy(g_ref.at[sslc(jn)], gbuf.at[b], gsem[b])
        return carry
    lax.fori_loop(0, ch // nbuf, body, 0)
    plsc.subcore_barrier()

    nt = rt // rc

    def pull(t, sem):
        return pltpu.make_async_copy(
            acc.at[pl.ds(s * rt + t * rc, rc)], cbuf.at[t % 2], sem)

    def push(t, sem):
        return pltpu.make_async_copy(
            cbuf.at[t % 2], out_ref.at[c, pl.ds(s * rt + t * rc, rc)], sem)

    pull(0, gsem[0]).start()
    for t in range(nt):
        b = t % 2
        pull(t, gsem[b]).wait()
        push(t, ssem[b]).start()
        if t + 1 < nt:
            nb = (t + 1) % 2
            if t >= 1:
                push(t - 1, ssem[nb]).wait()
            pull(t + 1, gsem[nb]).start()
    if nt >= 2:
        push(nt - 2, ssem[(nt - 2) % 2]).wait()
    push(nt - 1, ssem[(nt - 1) % 2]).wait()


def kernel(x, edge_index, W1, b1, W2, b2, W3, b3, attn_W, attn_b, fc_W, fc_b):
    N, d_in = x.shape
    E = edge_index.shape[1]
    H = W1.shape[1]

    epw = E // NW
    ept = E // NS
    cs = 80
    ch = epw // cs
    chd = ept // C
    nbuf = 5
    npad = ((N + 128 * NS - 1) // (128 * NS)) * (128 * NS)
    rt = npad // NS
    rc = 128
    dt = npad // NS

    mesh = plsc.VectorSubcoreMesh(core_axis_name="c", subcore_axis_name="s")

    deg_call = pl.kernel(
        functools.partial(_deg_body, E, ept, chd, dt),
        out_type=jax.ShapeDtypeStruct((npad,), jnp.float32),
        mesh=mesh,
        scratch_types=[
            pltpu.VMEM_SHARED((npad,), jnp.float32),
            pltpu.VMEM((ept,), jnp.int32),
            pltpu.VMEM((C,), jnp.float32),
            pltpu.VMEM((dt,), jnp.float32),
        ] + [pltpu.SemaphoreType.DMA] * 5,
        compiler_params=pltpu.CompilerParams(use_tc_tiling_on_sc=False),
    )
    spmm_call = pl.kernel(
        functools.partial(_spmm_body, E, epw, cs, ch, rt, rc, H, nbuf),
        out_type=jax.ShapeDtypeStruct((NC, npad, H), jnp.float32),
        mesh=mesh,
        scratch_types=[
            pltpu.VMEM_SHARED((npad, H), jnp.float32),
            pltpu.VMEM((epw,), jnp.int32),
            pltpu.VMEM((epw,), jnp.int32),
            pltpu.VMEM((nbuf, cs, H), jnp.float32),
            pltpu.VMEM((2, rc, H), jnp.float32),
        ] + [pltpu.SemaphoreType.DMA] * (2 * nbuf),
        compiler_params=pltpu.CompilerParams(use_tc_tiling_on_sc=False),
    )

    m = N // 2
    ei2 = (edge_index * 2
           - jnp.where(edge_index >= m, 2 * m - 1, 0)).reshape(2 * E)

    deg_p = deg_call(ei2)
    degm = deg_p[:N].reshape(m, 2)

    t1 = pl.pallas_call(
        functools.partial(_t1_body, N, H),
        out_shape=(jax.ShapeDtypeStruct((m, 2 * H), jnp.float32),
                   jax.ShapeDtypeStruct((m, 2 * H), jnp.float32)),
    )
    dis, g = t1(degm, x, W1)

    t2 = pl.pallas_call(
        functools.partial(_t2_body, N, npad),
        out_shape=jax.ShapeDtypeStruct((m, 2 * H), jnp.float32),
    )
    for b, wn in ((b1, W2), (b2, W3)):
        sp = spmm_call(ei2, g.reshape(N, H))
        g = t2(sp.reshape(NC * npad // 2, 2 * H), g, dis,
               b.reshape(1, H), wn)

    sp = spmm_call(ei2, g.reshape(N, H))
    t3 = pl.pallas_call(
        functools.partial(_t3_body, N, npad, H),
        out_shape=(jax.ShapeDtypeStruct((m, 2), jnp.float32),
                   jax.ShapeDtypeStruct((m, 2), jnp.float32)),
    )
    o2, a2 = t3(sp.reshape(NC * npad // 2, 2 * H), g, dis,
                b3.reshape(1, H), attn_W, attn_b.reshape(1, 1),
                fc_W, fc_b.reshape(1, 1))
    out = o2.T.reshape(N, 1)
    attn = a2.T.reshape(N, 1)
    return out, attn
